# Initial kernel scaffold; baseline (speedup 1.0000x reference)
import jax
import jax.numpy as jnp
from jax.experimental import pallas as pl


def kernel(x, edge_index, W1, b1, W2, b2, W3, b3, W4, b4):
    raise NotImplementedError("write your pallas kernel here")


# speedup vs baseline: 25.6045x; 25.6045x over previous
import functools

import jax
import jax.numpy as jnp
from jax import lax
from jax.experimental import pallas as pl
from jax.experimental.pallas import tpu as pltpu
from jax.experimental.pallas import tpu_sc as plsc

N = 10000
NP = 10240
E = 320000
EP = 327680
NW = 32
EPT = EP // NW
K = 128
NB = EPT // K
NC, NS = 2, 16
SLICE = NP // NS

_mesh = plsc.VectorSubcoreMesh(core_axis_name="c", subcore_axis_name="s",
                               num_cores=NC, num_subcores=NS)


@functools.partial(
    pl.kernel,
    out_type=(jax.ShapeDtypeStruct((NC, NP), jnp.float32),
              jax.ShapeDtypeStruct((NC, NP), jnp.float32)),
    mesh=_mesh,
    scratch_types=[
        pltpu.VMEM_SHARED((NP,), jnp.float32),
        pltpu.VMEM_SHARED((NP,), jnp.float32),
        pltpu.VMEM((K,), jnp.int32),
        pltpu.VMEM((K,), jnp.int32),
        pltpu.VMEM((K,), jnp.float32),
        pltpu.VMEM((K,), jnp.float32),
        pltpu.VMEM((SLICE,), jnp.float32),
    ],
)
def _degree_kernel(rows_hbm, cols_hbm, cnt_hbm, nself_hbm,
                   cnt_sh, nself_sh, ridx, cidx, ones_v, self_v, zeros_v):
    c = lax.axis_index("c")
    s = lax.axis_index("s")
    zero16 = jnp.zeros((16,), jnp.float32)
    one16 = jnp.ones((16,), jnp.float32)
    for j in range(SLICE // 16):
        zeros_v[pl.ds(j * 16, 16)] = zero16
    for j in range(K // 16):
        ones_v[pl.ds(j * 16, 16)] = one16
    pltpu.sync_copy(zeros_v, cnt_sh.at[pl.ds(s * SLICE, SLICE)])
    pltpu.sync_copy(zeros_v, nself_sh.at[pl.ds(s * SLICE, SLICE)])
    plsc.subcore_barrier()
    base = (c * NS + s) * EPT

    def body(b, carry):
        off = base + b * K
        pltpu.sync_copy(rows_hbm.at[pl.ds(off, K)], ridx)
        pltpu.sync_copy(cols_hbm.at[pl.ds(off, K)], cidx)
        for j in range(K // 16):
            rv = ridx[pl.ds(j * 16, 16)]
            cv = cidx[pl.ds(j * 16, 16)]
            self_v[pl.ds(j * 16, 16)] = jnp.where(rv == cv, 1.0, 0.0)
        pltpu.sync_copy(ones_v, cnt_sh.at[cidx], add=True)
        pltpu.sync_copy(self_v, nself_sh.at[cidx], add=True)
        return carry

    lax.fori_loop(0, NB, body, 0)
    plsc.subcore_barrier()
    pltpu.sync_copy(cnt_sh.at[pl.ds(s * SLICE, SLICE)],
                    cnt_hbm.at[c, pl.ds(s * SLICE, SLICE)])
    pltpu.sync_copy(nself_sh.at[pl.ds(s * SLICE, SLICE)],
                    nself_hbm.at[c, pl.ds(s * SLICE, SLICE)])


def _make_propagate(F):
    @functools.partial(
        pl.kernel,
        out_type=jax.ShapeDtypeStruct((NC, NP, F), jnp.float32),
        mesh=_mesh,
        scratch_types=[
            pltpu.VMEM_SHARED((NP, F), jnp.float32),
            pltpu.VMEM((K,), jnp.int32),
            pltpu.VMEM((K,), jnp.int32),
            pltpu.VMEM((K,), jnp.int32),
            pltpu.VMEM((K,), jnp.int32),
            pltpu.VMEM((K, F), jnp.float32),
            pltpu.VMEM((K, F), jnp.float32),
            pltpu.SemaphoreType.DMA,
            pltpu.SemaphoreType.DMA,
            pltpu.SemaphoreType.DMA,
            pltpu.SemaphoreType.DMA,
        ],
    )
    def prop_k(h_hbm, rows_hbm, cols_hbm, acc_hbm, acc_sh,
               ridx0, ridx1, cidx0, cidx1, gbuf0, gbuf1,
               gsem0, gsem1, ssem0, ssem1):
        c = lax.axis_index("c")
        s = lax.axis_index("s")
        zero16 = jnp.zeros((16,), jnp.float32)

        def zrow(r, carry):
            for j in range(F // 16):
                gbuf0[r, pl.ds(j * 16, 16)] = zero16
            return carry

        lax.fori_loop(0, K, zrow, 0)
        for k in range(SLICE // K):
            pltpu.sync_copy(gbuf0, acc_sh.at[pl.ds(s * SLICE + k * K, K)])
        plsc.subcore_barrier()

        base = (c * NS + s) * EPT

        def load_idx(b, ridx, cidx):
            off = base + b * K
            pltpu.sync_copy(rows_hbm.at[pl.ds(off, K)], ridx)
            pltpu.sync_copy(cols_hbm.at[pl.ds(off, K)], cidx)

        load_idx(0, ridx0, cidx0)
        pltpu.async_copy(h_hbm.at[ridx0], gbuf0, gsem0)
        load_idx(1, ridx1, cidx1)
        pltpu.async_copy(h_hbm.at[ridx1], gbuf1, gsem1)

        bufs = ((ridx0, cidx0, gbuf0, gsem0, ssem0),
                (ridx1, cidx1, gbuf1, gsem1, ssem1))

        def step(b, i):
            ridx, cidx, gbuf, gsem, ssem = bufs[i]
            pltpu.make_async_copy(h_hbm.at[ridx], gbuf, gsem).wait()
            pltpu.async_copy(gbuf, acc_sh.at[cidx], ssem, add=True)
            pltpu.make_async_copy(gbuf, acc_sh.at[cidx], ssem).wait()

            @pl.when(b + 2 < NB)
            def _():
                load_idx(b + 2, ridx, cidx)
                pltpu.async_copy(h_hbm.at[ridx], gbuf, gsem)

        def body(g, carry):
            step(2 * g, 0)
            step(2 * g + 1, 1)
            return carry

        lax.fori_loop(0, NB // 2, body, 0)
        plsc.subcore_barrier()
        pltpu.sync_copy(acc_sh.at[pl.ds(s * SLICE, SLICE)],
                        acc_hbm.at[c, pl.ds(s * SLICE, SLICE)])

    return prop_k


_prop128 = _make_propagate(128)


def _scales(cnt2, nself2):
    def body(cnt_ref, ns_ref, dss_ref, cas_ref, dsh_ref, cah_ref):
        cnt = cnt_ref[0:1, :] + cnt_ref[1:2, :]
        ns = ns_ref[0:1, :] + ns_ref[1:2, :]
        nonself = cnt - ns
        ds_s = lax.rsqrt(nonself + 1.0)
        has = ns > 0.5
        ds_h = lax.rsqrt(nonself + jnp.where(has, 1.0, 2.0))
        dss_ref[...] = ds_s
        cas_ref[...] = ds_s * (1.0 - ns)
        dsh_ref[...] = ds_h
        cah_ref[...] = ds_h * (jnp.where(has, -1.0, 2.0) + ns)

    shp = jax.ShapeDtypeStruct((1, NP), jnp.float32)
    outs = pl.pallas_call(
        body,
        in_specs=[pl.BlockSpec((NC, NP), lambda: (0, 0)),
                  pl.BlockSpec((NC, NP), lambda: (0, 0))],
        out_specs=[pl.BlockSpec((1, NP), lambda: (0, 0))] * 4,
        out_shape=[shp] * 4,
    )(cnt2, nself2)
    return [o.reshape(NP, 1) for o in outs]


_BM = 1024


def _matmul_first(x, W, ds):
    Fout, Fin = W.shape

    def body(x_ref, w_ref, ds_ref, o_ref):
        h = lax.dot_general(x_ref[...], w_ref[...], (((1,), (1,)), ((), ())),
                            preferred_element_type=jnp.float32)
        o_ref[...] = h * ds_ref[...]

    return pl.pallas_call(
        body,
        grid=(NP // _BM,),
        in_specs=[pl.BlockSpec((_BM, Fin), lambda i: (i, 0)),
                  pl.BlockSpec((Fout, Fin), lambda i: (0, 0)),
                  pl.BlockSpec((_BM, 1), lambda i: (i, 0))],
        out_specs=pl.BlockSpec((_BM, Fout), lambda i: (i, 0)),
        out_shape=jax.ShapeDtypeStruct((NP, Fout), jnp.float32),
    )(x, W, ds)


def _combine_matmul(acc, hprev, W, brow, ds_signed, ca, ds_new):
    Fout, Fin = W.shape

    def body(acc_ref, h_ref, w_ref, b_ref, dsp_ref, cap_ref, dsn_ref, o_ref):
        xin = (dsp_ref[...] * (acc_ref[0] + acc_ref[1])
               + cap_ref[...] * h_ref[...] + b_ref[...])
        xin = jnp.maximum(xin, 0.0)
        h = lax.dot_general(xin, w_ref[...], (((1,), (1,)), ((), ())),
                            preferred_element_type=jnp.float32)
        o_ref[...] = h * dsn_ref[...]

    return pl.pallas_call(
        body,
        grid=(NP // _BM,),
        in_specs=[pl.BlockSpec((NC, _BM, Fin), lambda i: (0, i, 0)),
                  pl.BlockSpec((_BM, Fin), lambda i: (i, 0)),
                  pl.BlockSpec((Fout, Fin), lambda i: (0, 0)),
                  pl.BlockSpec((1, Fin), lambda i: (0, 0)),
                  pl.BlockSpec((_BM, 1), lambda i: (i, 0)),
                  pl.BlockSpec((_BM, 1), lambda i: (i, 0)),
                  pl.BlockSpec((_BM, 1), lambda i: (i, 0))],
        out_specs=pl.BlockSpec((_BM, Fout), lambda i: (i, 0)),
        out_shape=jax.ShapeDtypeStruct((NP, Fout), jnp.float32),
    )(acc, hprev, W, brow, ds_signed, ca, ds_new)


def _final_combine(acc, hprev, brow, ds_signed, ca):
    Fin = hprev.shape[1]

    def body(acc_ref, h_ref, b_ref, dsp_ref, cap_ref, o_ref):
        o_ref[...] = (dsp_ref[...] * (acc_ref[0] + acc_ref[1])
                      + cap_ref[...] * h_ref[...] + b_ref[...])

    return pl.pallas_call(
        body,
        grid=(NP // _BM,),
        in_specs=[pl.BlockSpec((NC, _BM, Fin), lambda i: (0, i, 0)),
                  pl.BlockSpec((_BM, Fin), lambda i: (i, 0)),
                  pl.BlockSpec((1, Fin), lambda i: (0, 0)),
                  pl.BlockSpec((_BM, 1), lambda i: (i, 0)),
                  pl.BlockSpec((_BM, 1), lambda i: (i, 0))],
        out_specs=pl.BlockSpec((_BM, Fin), lambda i: (i, 0)),
        out_shape=jax.ShapeDtypeStruct((NP, Fin), jnp.float32),
    )(acc, hprev, brow, ds_signed, ca)


def _pad_w(W):
    Fout, Fin = W.shape
    return jnp.pad(W, ((0, 128 - Fout), (0, 128 - Fin)))


def _pad_b(b):
    return jnp.pad(b, (0, 128 - b.shape[0])).reshape(1, 128)


def kernel(x, edge_index, W1, b1, W2, b2, W3, b3, W4, b4):
    ei = edge_index.astype(jnp.int32)
    rows, cols = ei[0], ei[1]
    npad = EP - E
    ar = jnp.arange(npad, dtype=jnp.int32)
    rows_p = jnp.concatenate([rows, (ar * 37) % N])
    cols_p = jnp.concatenate([cols, N + (ar % 192)])
    x_p = jnp.pad(x, ((0, NP - N), (0, 0)))

    cnt2, nself2 = _degree_kernel(rows_p, cols_p)
    ds_s, ca_s, ds_h, ca_h = _scales(cnt2, nself2)
    W1p, W2p, W3p, W4p = (_pad_w(W) for W in (W1, W2, W3, W4))
    b1r, b2r, b3r, b4r = (_pad_b(b) for b in (b1, b2, b3, b4))

    h1 = _matmul_first(x_p, W1p, ds_s)
    a1 = _prop128(h1, rows_p, cols_p)
    h2 = _combine_matmul(a1, h1, W2p, b1r, ds_s, ca_s, ds_s)
    a2 = _prop128(h2, rows_p, cols_p)
    h3 = _combine_matmul(a2, h2, W3p, b2r, ds_s, ca_s, ds_h)
    a3 = _prop128(h3, rows_p, cols_p)
    h4 = _combine_matmul(a3, h3, W4p, b3r, -ds_h, ca_h, ds_h)
    a4 = _prop128(h4, rows_p, cols_p)
    out = _final_combine(a4, h4, b4r, -ds_h, ca_h)
    return out[:N]


# speedup vs baseline: 31.1609x; 1.2170x over previous
import functools

import jax
import jax.numpy as jnp
from jax import lax
from jax.experimental import pallas as pl
from jax.experimental.pallas import tpu as pltpu
from jax.experimental.pallas import tpu_sc as plsc

N = 10000
NP = 10240
E = 320000
EP = 327680
NW = 32
EPT = EP // NW
K = 128
NB = EPT // K
NC, NS = 2, 16
SLICE = NP // NS

_mesh = plsc.VectorSubcoreMesh(core_axis_name="c", subcore_axis_name="s",
                               num_cores=NC, num_subcores=NS)


@functools.partial(
    pl.kernel,
    out_type=(jax.ShapeDtypeStruct((NC, NP), jnp.float32),
              jax.ShapeDtypeStruct((NC, NP), jnp.float32)),
    mesh=_mesh,
    scratch_types=[
        pltpu.VMEM_SHARED((NP,), jnp.float32),
        pltpu.VMEM_SHARED((NP,), jnp.float32),
        pltpu.VMEM((2, K), jnp.int32),
        pltpu.VMEM((2, K), jnp.int32),
        pltpu.VMEM((K,), jnp.float32),
        pltpu.VMEM((K,), jnp.float32),
        pltpu.VMEM((K,), jnp.float32),
        pltpu.VMEM((SLICE,), jnp.float32),
        pltpu.SemaphoreType.DMA,
        pltpu.SemaphoreType.DMA,
        pltpu.SemaphoreType.DMA,
        pltpu.SemaphoreType.DMA,
    ],
)
def _degree_kernel(eidx_hbm, cnt_hbm, nself_hbm,
                   cnt_sh, nself_sh, idx0, idx1, ones_v, self0, self1,
                   zeros_v, isem0, isem1, ssem0, ssem1):
    c = lax.axis_index("c")
    s = lax.axis_index("s")
    zero16 = jnp.zeros((16,), jnp.float32)
    one16 = jnp.ones((16,), jnp.float32)
    for j in range(SLICE // 16):
        zeros_v[pl.ds(j * 16, 16)] = zero16
    for j in range(K // 16):
        ones_v[pl.ds(j * 16, 16)] = one16
    pltpu.sync_copy(zeros_v, cnt_sh.at[pl.ds(s * SLICE, SLICE)])
    pltpu.sync_copy(zeros_v, nself_sh.at[pl.ds(s * SLICE, SLICE)])
    plsc.subcore_barrier()
    base = (c * NS + s) * (EPT // K)
    bufs = ((idx0, self0, isem0, ssem0), (idx1, self1, isem1, ssem1))

    pltpu.async_copy(eidx_hbm.at[base], idx0, isem0)
    pltpu.async_copy(eidx_hbm.at[base + 1], idx1, isem1)

    def step(b, i):
        idx, self_v, isem, ssem = bufs[i]
        pltpu.make_async_copy(eidx_hbm.at[base], idx, isem).wait()
        for j in range(K // 16):
            rv = idx[0, pl.ds(j * 16, 16)]
            cv = idx[1, pl.ds(j * 16, 16)]
            self_v[pl.ds(j * 16, 16)] = jnp.where(rv == cv, 1.0, 0.0)
        pltpu.async_copy(ones_v, cnt_sh.at[idx.at[1]], ssem, add=True)
        pltpu.async_copy(self_v, nself_sh.at[idx.at[1]], ssem, add=True)
        pltpu.make_async_copy(ones_v, cnt_sh.at[idx.at[1]], ssem).wait()
        pltpu.make_async_copy(self_v, nself_sh.at[idx.at[1]], ssem).wait()

        @pl.when(b + 2 < NB)
        def _():
            pltpu.async_copy(eidx_hbm.at[base + b + 2], idx, isem)

    def body(g, carry):
        step(2 * g, 0)
        step(2 * g + 1, 1)
        return carry

    lax.fori_loop(0, NB // 2, body, 0)
    plsc.subcore_barrier()
    pltpu.sync_copy(cnt_sh.at[pl.ds(s * SLICE, SLICE)],
                    cnt_hbm.at[c, pl.ds(s * SLICE, SLICE)])
    pltpu.sync_copy(nself_sh.at[pl.ds(s * SLICE, SLICE)],
                    nself_hbm.at[c, pl.ds(s * SLICE, SLICE)])


def _make_propagate(F):
    RD = 2
    RI = 4

    @functools.partial(
        pl.kernel,
        out_type=jax.ShapeDtypeStruct((NC, NP, F), jnp.float32),
        mesh=_mesh,
        scratch_types=[
            pltpu.VMEM_SHARED((NP, F), jnp.float32),
            [pltpu.VMEM((2, K), jnp.int32) for _ in range(RI)],
            [pltpu.VMEM((K, F), jnp.float32) for _ in range(RD)],
            [pltpu.SemaphoreType.DMA for _ in range(RI)],
            [pltpu.SemaphoreType.DMA for _ in range(RD)],
            [pltpu.SemaphoreType.DMA for _ in range(RD)],
        ],
    )
    def prop_k(h_hbm, eidx_hbm, acc_hbm, acc_sh, idx, gbuf, isem, gsem, ssem):
        c = lax.axis_index("c")
        s = lax.axis_index("s")
        zero16 = jnp.zeros((16,), jnp.float32)

        def zrow(r, carry):
            for j in range(F // 16):
                gbuf[0][r, pl.ds(j * 16, 16)] = zero16
            return carry

        lax.fori_loop(0, K, zrow, 0)
        for k in range(SLICE // K):
            pltpu.sync_copy(gbuf[0], acc_sh.at[pl.ds(s * SLICE + k * K, K)])
        plsc.subcore_barrier()

        base = (c * NS + s) * NB

        def start_idx(b, i):
            pltpu.async_copy(eidx_hbm.at[b + base], idx[i], isem[i])

        def wait_idx(i):
            pltpu.make_async_copy(eidx_hbm.at[base], idx[i], isem[i]).wait()

        def start_gather(i4, i2):
            pltpu.async_copy(h_hbm.at[idx[i4].at[0]], gbuf[i2], gsem[i2])

        def wait_gather(i4, i2):
            pltpu.make_async_copy(h_hbm.at[idx[i4].at[0]], gbuf[i2],
                                  gsem[i2]).wait()

        def start_scatter(i4, i2):
            pltpu.async_copy(gbuf[i2], acc_sh.at[idx[i4].at[1]], ssem[i2],
                             add=True)

        def wait_scatter(i4, i2):
            pltpu.make_async_copy(gbuf[i2], acc_sh.at[idx[i4].at[1]],
                                  ssem[i2]).wait()

        start_idx(0, 0)
        start_idx(1, 1)
        wait_idx(0)
        start_gather(0, 0)

        def step(b, i4, swait, istart, gstart):
            i2 = i4 % RD
            wait_gather(i4, i2)
            start_scatter(i4, i2)
            if swait:
                wait_scatter((i4 + RI - 1) % RI, (i2 + 1) % RD)
            if gstart:
                wait_idx((i4 + 1) % RI)
                start_gather((i4 + 1) % RI, (i2 + 1) % RD)
            if istart:
                start_idx(b + 2, (i4 + 2) % RI)

        step(0, 0, False, True, True)
        step(1, 1, True, True, True)

        def body(g, carry):
            b0 = 2 + 4 * g
            for k in range(4):
                step(b0 + k, (2 + k) % RI, True, True, True)
            return carry

        lax.fori_loop(0, (NB - 4) // 4, body, 0)
        step(NB - 2, (NB - 2) % RI, True, False, True)
        step(NB - 1, (NB - 1) % RI, True, False, False)
        wait_scatter((NB - 1) % RI, (NB - 1) % RD)
        plsc.subcore_barrier()
        pltpu.sync_copy(acc_sh.at[pl.ds(s * SLICE, SLICE)],
                        acc_hbm.at[c, pl.ds(s * SLICE, SLICE)])

    return prop_k


_prop128 = _make_propagate(128)


def _scales(cnt2, nself2):
    def body(cnt_ref, ns_ref, dss_ref, cas_ref, dsh_ref, cah_ref):
        cnt = cnt_ref[0:1, :] + cnt_ref[1:2, :]
        ns = ns_ref[0:1, :] + ns_ref[1:2, :]
        nonself = cnt - ns
        ds_s = lax.rsqrt(nonself + 1.0)
        has = ns > 0.5
        ds_h = lax.rsqrt(nonself + jnp.where(has, 1.0, 2.0))
        dss_ref[...] = ds_s
        cas_ref[...] = ds_s * (1.0 - ns)
        dsh_ref[...] = ds_h
        cah_ref[...] = ds_h * (jnp.where(has, -1.0, 2.0) + ns)

    shp = jax.ShapeDtypeStruct((1, NP), jnp.float32)
    outs = pl.pallas_call(
        body,
        in_specs=[pl.BlockSpec((NC, NP), lambda: (0, 0)),
                  pl.BlockSpec((NC, NP), lambda: (0, 0))],
        out_specs=[pl.BlockSpec((1, NP), lambda: (0, 0))] * 4,
        out_shape=[shp] * 4,
    )(cnt2, nself2)
    return [o.reshape(NP, 1) for o in outs]


_BM = 1024


def _matmul_first(x, W, ds):
    Fout, Fin = W.shape

    def body(x_ref, w_ref, ds_ref, o_ref):
        h = lax.dot_general(x_ref[...], w_ref[...], (((1,), (1,)), ((), ())),
                            preferred_element_type=jnp.float32)
        o_ref[...] = h * ds_ref[...]

    return pl.pallas_call(
        body,
        grid=(NP // _BM,),
        in_specs=[pl.BlockSpec((_BM, Fin), lambda i: (i, 0)),
                  pl.BlockSpec((Fout, Fin), lambda i: (0, 0)),
                  pl.BlockSpec((_BM, 1), lambda i: (i, 0))],
        out_specs=pl.BlockSpec((_BM, Fout), lambda i: (i, 0)),
        out_shape=jax.ShapeDtypeStruct((NP, Fout), jnp.float32),
    )(x, W, ds)


def _combine_matmul(acc, hprev, W, brow, ds_signed, ca, ds_new):
    Fout, Fin = W.shape

    def body(acc_ref, h_ref, w_ref, b_ref, dsp_ref, cap_ref, dsn_ref, o_ref):
        xin = (dsp_ref[...] * (acc_ref[0] + acc_ref[1])
               + cap_ref[...] * h_ref[...] + b_ref[...])
        xin = jnp.maximum(xin, 0.0)
        h = lax.dot_general(xin, w_ref[...], (((1,), (1,)), ((), ())),
                            preferred_element_type=jnp.float32)
        o_ref[...] = h * dsn_ref[...]

    return pl.pallas_call(
        body,
        grid=(NP // _BM,),
        in_specs=[pl.BlockSpec((NC, _BM, Fin), lambda i: (0, i, 0)),
                  pl.BlockSpec((_BM, Fin), lambda i: (i, 0)),
                  pl.BlockSpec((Fout, Fin), lambda i: (0, 0)),
                  pl.BlockSpec((1, Fin), lambda i: (0, 0)),
                  pl.BlockSpec((_BM, 1), lambda i: (i, 0)),
                  pl.BlockSpec((_BM, 1), lambda i: (i, 0)),
                  pl.BlockSpec((_BM, 1), lambda i: (i, 0))],
        out_specs=pl.BlockSpec((_BM, Fout), lambda i: (i, 0)),
        out_shape=jax.ShapeDtypeStruct((NP, Fout), jnp.float32),
    )(acc, hprev, W, brow, ds_signed, ca, ds_new)


def _final_combine(acc, hprev, brow, ds_signed, ca):
    Fin = hprev.shape[1]

    def body(acc_ref, h_ref, b_ref, dsp_ref, cap_ref, o_ref):
        o_ref[...] = (dsp_ref[...] * (acc_ref[0] + acc_ref[1])
                      + cap_ref[...] * h_ref[...] + b_ref[...])

    return pl.pallas_call(
        body,
        grid=(NP // _BM,),
        in_specs=[pl.BlockSpec((NC, _BM, Fin), lambda i: (0, i, 0)),
                  pl.BlockSpec((_BM, Fin), lambda i: (i, 0)),
                  pl.BlockSpec((1, Fin), lambda i: (0, 0)),
                  pl.BlockSpec((_BM, 1), lambda i: (i, 0)),
                  pl.BlockSpec((_BM, 1), lambda i: (i, 0))],
        out_specs=pl.BlockSpec((_BM, Fin), lambda i: (i, 0)),
        out_shape=jax.ShapeDtypeStruct((NP, Fin), jnp.float32),
    )(acc, hprev, brow, ds_signed, ca)


def _pad_w(W):
    Fout, Fin = W.shape
    return jnp.pad(W, ((0, 128 - Fout), (0, 128 - Fin)))


def _pad_b(b):
    return jnp.pad(b, (0, 128 - b.shape[0])).reshape(1, 128)


def kernel(x, edge_index, W1, b1, W2, b2, W3, b3, W4, b4):
    ei = edge_index.astype(jnp.int32)
    rows, cols = ei[0], ei[1]
    npad = EP - E
    ar = jnp.arange(npad, dtype=jnp.int32)
    rows_p = jnp.concatenate([rows, (ar * 37) % N])
    cols_p = jnp.concatenate([cols, N + (ar % 192)])
    eidx = jnp.stack([rows_p.reshape(EP // K, K),
                      cols_p.reshape(EP // K, K)], axis=1)
    x_p = jnp.pad(x, ((0, NP - N), (0, 0)))

    cnt2, nself2 = _degree_kernel(eidx)
    ds_s, ca_s, ds_h, ca_h = _scales(cnt2, nself2)
    W1p, W2p, W3p, W4p = (_pad_w(W) for W in (W1, W2, W3, W4))
    b1r, b2r, b3r, b4r = (_pad_b(b) for b in (b1, b2, b3, b4))

    h1 = _matmul_first(x_p, W1p, ds_s)
    a1 = _prop128(h1, eidx)
    h2 = _combine_matmul(a1, h1, W2p, b1r, ds_s, ca_s, ds_s)
    a2 = _prop128(h2, eidx)
    h3 = _combine_matmul(a2, h2, W3p, b2r, ds_s, ca_s, ds_h)
    a3 = _prop128(h3, eidx)
    h4 = _combine_matmul(a3, h3, W4p, b3r, -ds_h, ca_h, ds_h)
    a4 = _prop128(h4, eidx)
    out = _final_combine(a4, h4, b4r, -ds_h, ca_h)
    return out[:N]
